# baseline (speedup 1.0000x reference)
import functools

import jax
import jax.numpy as jnp
from jax import lax
from jax.experimental import pallas as pl
from jax.experimental.pallas import tpu as pltpu
from jax.experimental.pallas import tpu_sc as plsc

_S, _K, _A, _T, _C = 32, 64, 64, 80, 3
_KP = 6
_NMS_THRESH = (2.5, 1.0, 2.0)
_ROW = _T * _C
_KEEP = 16 * _C
_B = _S * _A * _KP
_NW = 32
_CHUNK = 128
_NCH = _B // (_NW * _CHUNK)
_R = _S * _K * _A
_RW = _R // _NW
_ECH = _RW // _CHUNK

_SC_PARAMS = pltpu.CompilerParams(
    use_tc_tiling_on_sc=False, needs_layout_passes=False)


def _sc_extract_body(table16_hbm, out_hbm, idx_v, rows_v, xy_v, sem):
    wid = lax.axis_index("s") * 2 + lax.axis_index("c")
    lane = lax.iota(jnp.int32, 16)
    base = wid * _RW
    for ch in range(_ECH):
        for u in range(_CHUNK // 16):
            idx_v[ch, pl.ds(u * 16, 16)] = (
                (base + ch * _CHUNK + u * 16 + lane) * 15 + 14)
    copies = [
        pltpu.async_copy(table16_hbm.at[idx_v.at[ch]], rows_v.at[ch], sem)
        for ch in range(_ECH)
    ]
    for cp in copies:
        cp.wait()
    w13 = jnp.full((16,), 13, jnp.int32)
    w14 = jnp.full((16,), 14, jnp.int32)
    for ch in range(_ECH):
        chf = jnp.full((16,), ch, jnp.int32)

        def body(g, carry, chf=chf):
            rf = g * 16 + lane
            xv = plsc.load_gather(rows_v, [chf, rf, w13])
            yv = plsc.load_gather(rows_v, [chf, rf, w14])
            xy_v[0, ch, pl.ds(g * 16, 16)] = xv
            xy_v[1, ch, pl.ds(g * 16, 16)] = yv
            return carry

        lax.fori_loop(0, _CHUNK // 16, body, 0)
    pltpu.sync_copy(xy_v, out_hbm.at[wid])


def _nms_body(sc_ref, xs_ref, ys_ref, agt_ref, fidx_ref, sout_ref):
    s = pl.program_id(0)
    sc_raw = sc_ref[0]
    xs = xs_ref[0]
    ys = ys_ref[0]
    agt = agt_ref[0]
    thresh = (_NMS_THRESH[0] * agt[0:1, :]
              + _NMS_THRESH[1] * agt[1:2, :]
              + _NMS_THRESH[2] * agt[2:3, :])

    m = jnp.max(sc_raw, axis=0, keepdims=True)
    e = jnp.exp(sc_raw - m)
    p = e / jnp.sum(e, axis=0, keepdims=True)

    kiota = lax.broadcasted_iota(jnp.int32, (_K, _A), 0)
    aiota = lax.broadcasted_iota(jnp.int32, (1, _A), 1)

    scn = p
    psel = []
    for j in range(_KP):
        mx = jnp.max(scn, axis=0, keepdims=True)
        idx = jnp.min(jnp.where(scn == mx, kiota, _K), axis=0, keepdims=True)
        oh = kiota == idx
        xsel = jnp.sum(jnp.where(oh, xs, 0.0), axis=0, keepdims=True)
        ysel = jnp.sum(jnp.where(oh, ys, 0.0), axis=0, keepdims=True)
        psel.append(jnp.sum(jnp.where(oh, p, 0.0), axis=0, keepdims=True))
        dx = xs - xsel
        dy = ys - ysel
        drow = jnp.sqrt(dx * dx + dy * dy)
        within = drow < thresh
        scn = scn * jnp.where(within, 0.01, 1.0)
        scn = jnp.where(oh, -1.0, scn)
        fidx_ref[0, j:j + 1, :] = s * (_K * _A) + idx * _A + aiota

    pm = psel[0]
    for j in range(1, _KP):
        pm = jnp.maximum(pm, psel[j])
    r2 = [(pj / pm) * (pj / pm) for pj in psel]
    tot = r2[0]
    for j in range(1, _KP):
        tot = tot + r2[j]
    for j in range(_KP):
        sout_ref[0, j:j + 1, :] = r2[j] / tot


_nms_call = pl.pallas_call(
    _nms_body,
    grid=(_S,),
    in_specs=[
        pl.BlockSpec((1, _K, _A), lambda s: (s, 0, 0)),
        pl.BlockSpec((1, _K, _A), lambda s: (s, 0, 0)),
        pl.BlockSpec((1, _K, _A), lambda s: (s, 0, 0)),
        pl.BlockSpec((1, _C, _A), lambda s: (s, 0, 0)),
    ],
    out_specs=[
        pl.BlockSpec((1, _KP, _A), lambda s: (s, 0, 0)),
        pl.BlockSpec((1, _KP, _A), lambda s: (s, 0, 0)),
    ],
    out_shape=[
        jax.ShapeDtypeStruct((_S, _KP, _A), jnp.int32),
        jax.ShapeDtypeStruct((_S, _KP, _A), jnp.float32),
    ],
)


def _sc_gather_body(table_hbm, idx_hbm, out_hbm, idx_v, rows_v, out_v, sem):
    wid = lax.axis_index("s") * 2 + lax.axis_index("c")
    pltpu.sync_copy(idx_hbm.at[wid], idx_v)
    copies = [
        pltpu.async_copy(table_hbm.at[idx_v.at[j]], rows_v.at[j], sem)
        for j in range(_NCH)
    ]
    for cp in copies:
        cp.wait()

    lane = lax.iota(jnp.int32, 16)
    srcs = []
    for v in range(_C):
        pos = lane + v * 16
        c3 = pos % 3
        t5 = (pos - c3) // 3
        srcs.append(12 + 15 * t5 + c3)

    for j in range(_NCH):
        jf = jnp.full((16,), j, jnp.int32)

        def body(rl, carry, jf=jf):
            rf = jnp.full((16,), rl, jnp.int32)
            for v in range(_C):
                out_v[j, rl, pl.ds(v * 16, 16)] = plsc.load_gather(
                    rows_v, [jf, rf, srcs[v]])
            return carry

        lax.fori_loop(0, _CHUNK, body, 0)

    pltpu.sync_copy(out_v, out_hbm.at[wid])


@functools.cache
def _sc_calls():
    mesh = plsc.VectorSubcoreMesh(core_axis_name="c", subcore_axis_name="s")
    extract = functools.partial(
        pl.kernel,
        mesh=mesh,
        out_type=jax.ShapeDtypeStruct((_NW, 2, _ECH, _CHUNK), jnp.float32),
        compiler_params=_SC_PARAMS,
        scratch_types=[
            pltpu.VMEM((_ECH, _CHUNK), jnp.int32),
            pltpu.VMEM((_ECH, _CHUNK, 16), jnp.float32),
            pltpu.VMEM((2, _ECH, _CHUNK), jnp.float32),
            pltpu.SemaphoreType.DMA,
        ],
    )(_sc_extract_body)
    gather = functools.partial(
        pl.kernel,
        mesh=mesh,
        out_type=jax.ShapeDtypeStruct((_NW, _NCH, _CHUNK, _KEEP), jnp.float32),
        compiler_params=_SC_PARAMS,
        scratch_types=[
            pltpu.VMEM((_NCH, _CHUNK), jnp.int32),
            pltpu.VMEM((_NCH, _CHUNK, _ROW), jnp.float32),
            pltpu.VMEM((_NCH, _CHUNK, _KEEP), jnp.float32),
            pltpu.SemaphoreType.DMA,
        ],
    )(_sc_gather_body)
    return extract, gather


def kernel(ag_type, trajs, scores):
    extract, gather = _sc_calls()
    table16 = trajs.reshape(_R * 15, 16)
    xy = extract(table16)
    xs = xy[:, 0].reshape(_S, _K, _A)
    ys = xy[:, 1].reshape(_S, _K, _A)
    agt = jnp.swapaxes(ag_type, 1, 2)
    fidx, sout = _nms_call(scores, xs, ys, agt)
    scores_k = jnp.swapaxes(sout, 1, 2)
    flat_idx = jnp.transpose(fidx, (0, 2, 1)).reshape(_NW, _NCH, _CHUNK)
    table = trajs.reshape(_R, _ROW)
    rows = gather(table, flat_idx)
    trajs_out = rows.reshape(_S, _A, _KP, 16, _C)
    return trajs_out, scores_k


# speedup vs baseline: 1.0121x; 1.0121x over previous
import functools

import jax
import jax.numpy as jnp
from jax import lax
from jax.experimental import pallas as pl
from jax.experimental.pallas import tpu as pltpu
from jax.experimental.pallas import tpu_sc as plsc

_S, _K, _A, _T, _C = 32, 64, 64, 80, 3
_KP = 6
_NMS_THRESH = (2.5, 1.0, 2.0)
_ROW = _T * _C
_KEEP = 16 * _C
_B = _S * _A * _KP
_NW = 32
_R = _S * _K * _A
_NSEG = _R * _ROW // 128
_RW = _R // _NW
_ECH = _RW // 128
_GCH = 12
_GSEG = 96
_GROWS = _GSEG // 3
_RING = 4

_SC_PARAMS = pltpu.CompilerParams(
    use_tc_tiling_on_sc=False, needs_layout_passes=False)


def _sc_extract_body(table_hbm, out_hbm, idx_v, ring_v, xy_v, *sems):
    wid = lax.axis_index("s") * 2 + lax.axis_index("c")
    lane = lax.iota(jnp.int32, 16)
    base = wid * _RW
    for ch in range(_ECH):
        for u in range(8):
            r = base + ch * 128 + u * 16 + lane
            idx_v[ch, pl.ds(u * 16, 16)] = (r * 240 + 237) >> 7
    copies = {}
    for ch in range(_RING):
        copies[ch] = pltpu.async_copy(
            table_hbm.at[idx_v.at[ch]], ring_v.at[ch], sems[ch])
    patx = (((lane & 7) * 240 + 237) & 127)
    for ch in range(_ECH):
        copies.pop(ch).wait()
        b = ch % _RING
        bf = jnp.full((16,), b, jnp.int32)
        for u in range(8):
            rowv = u * 16 + lane
            xy_v[0, ch, pl.ds(u * 16, 16)] = plsc.load_gather(
                ring_v, [bf, rowv, patx])
            xy_v[1, ch, pl.ds(u * 16, 16)] = plsc.load_gather(
                ring_v, [bf, rowv, patx + 1])
        nxt = ch + _RING
        if nxt < _ECH:
            copies[nxt] = pltpu.async_copy(
                table_hbm.at[idx_v.at[nxt]], ring_v.at[b], sems[b])
    pltpu.sync_copy(xy_v, out_hbm.at[wid])


def _nms_body(sc_ref, xs_ref, ys_ref, agt_ref, seg_ref, sout_ref):
    s = pl.program_id(0)
    sc_raw = sc_ref[0]
    xs = xs_ref[0]
    ys = ys_ref[0]
    agt = agt_ref[0]
    thresh = (_NMS_THRESH[0] * agt[0:1, :]
              + _NMS_THRESH[1] * agt[1:2, :]
              + _NMS_THRESH[2] * agt[2:3, :])

    m = jnp.max(sc_raw, axis=0, keepdims=True)
    e = jnp.exp(sc_raw - m)
    p = e / jnp.sum(e, axis=0, keepdims=True)

    kiota = lax.broadcasted_iota(jnp.int32, (_K, _A), 0)
    aiota = lax.broadcasted_iota(jnp.int32, (1, _A), 1)

    scn = p
    psel = []
    for j in range(_KP):
        mx = jnp.max(scn, axis=0, keepdims=True)
        idx = jnp.min(jnp.where(scn == mx, kiota, _K), axis=0, keepdims=True)
        oh = kiota == idx
        xsel = jnp.sum(jnp.where(oh, xs, 0.0), axis=0, keepdims=True)
        ysel = jnp.sum(jnp.where(oh, ys, 0.0), axis=0, keepdims=True)
        psel.append(jnp.sum(jnp.where(oh, p, 0.0), axis=0, keepdims=True))
        dx = xs - xsel
        dy = ys - ysel
        drow = jnp.sqrt(dx * dx + dy * dy)
        within = drow < thresh
        scn = scn * jnp.where(within, 0.01, 1.0)
        scn = jnp.where(oh, -1.0, scn)
        r = s * (_K * _A) + idx * _A + aiota
        seg0 = lax.shift_right_logical(r * 15, 3)
        for v in range(3):
            seg_ref[0, 3 * j + v:3 * j + v + 1, :] = jnp.minimum(
                seg0 + v, _NSEG - 1)

    pm = psel[0]
    for j in range(1, _KP):
        pm = jnp.maximum(pm, psel[j])
    r2 = [(pj / pm) * (pj / pm) for pj in psel]
    tot = r2[0]
    for j in range(1, _KP):
        tot = tot + r2[j]
    for j in range(_KP):
        sout_ref[0, j:j + 1, :] = r2[j] / tot


_nms_call = pl.pallas_call(
    _nms_body,
    grid=(_S,),
    in_specs=[
        pl.BlockSpec((1, _K, _A), lambda s: (s, 0, 0)),
        pl.BlockSpec((1, _K, _A), lambda s: (s, 0, 0)),
        pl.BlockSpec((1, _K, _A), lambda s: (s, 0, 0)),
        pl.BlockSpec((1, _C, _A), lambda s: (s, 0, 0)),
    ],
    out_specs=[
        pl.BlockSpec((1, 3 * _KP, _A), lambda s: (s, 0, 0)),
        pl.BlockSpec((1, _KP, _A), lambda s: (s, 0, 0)),
    ],
    out_shape=[
        jax.ShapeDtypeStruct((_S, 3 * _KP, _A), jnp.int32),
        jax.ShapeDtypeStruct((_S, _KP, _A), jnp.float32),
    ],
)


def _sc_gather_body(table_hbm, seg_hbm, out_hbm, idx_v, ring_v, out_v, *sems):
    wid = lax.axis_index("s") * 2 + lax.axis_index("c")
    lane = lax.iota(jnp.int32, 16)
    pltpu.sync_copy(seg_hbm.at[wid], idx_v)
    copies = {}
    for ch in range(_RING):
        copies[ch] = pltpu.async_copy(
            table_hbm.at[idx_v.at[ch]], ring_v.at[ch], sems[ch])

    for ch in range(_GCH):
        copies.pop(ch).wait()
        b = ch % _RING
        bf = jnp.full((16,), b, jnp.int32)
        for g in range(_GROWS // 16):
            il = g * 16 + lane
            pos = wid * (_B // _NW) + ch * _GROWS + il
            a = (pos // _KP) % _A
            off = ((0 - a) & 7) * 16
            rowbase = il * (3 * 128) + off
            outbase = (ch * _GROWS + il) * _KEEP

            def tbody(t5, carry, rowbase=rowbase, outbase=outbase, bf=bf):
                src0 = rowbase + (12 + 15 * t5)
                dst0 = outbase + 3 * t5
                for c3 in range(_C):
                    sp = src0 + c3
                    dp = dst0 + c3
                    gval = plsc.load_gather(
                        ring_v,
                        [bf, lax.shift_right_logical(sp, 7), sp & 127])
                    plsc.store_scatter(
                        out_v,
                        [lax.shift_right_logical(dp, 7), dp & 127], gval)
                return carry

            lax.fori_loop(0, 16, tbody, 0)
        nxt = ch + _RING
        if nxt < _GCH:
            copies[nxt] = pltpu.async_copy(
                table_hbm.at[idx_v.at[nxt]], ring_v.at[b], sems[b])

    pltpu.sync_copy(out_v, out_hbm.at[wid])


@functools.cache
def _sc_calls():
    mesh = plsc.VectorSubcoreMesh(core_axis_name="c", subcore_axis_name="s")
    extract = functools.partial(
        pl.kernel,
        mesh=mesh,
        out_type=jax.ShapeDtypeStruct((_NW, 2, _ECH, 128), jnp.float32),
        compiler_params=_SC_PARAMS,
        scratch_types=[
            pltpu.VMEM((_ECH, 128), jnp.int32),
            pltpu.VMEM((_RING, 128, 128), jnp.float32),
            pltpu.VMEM((2, _ECH, 128), jnp.float32),
        ] + [pltpu.SemaphoreType.DMA] * _RING,
    )(_sc_extract_body)
    gather = functools.partial(
        pl.kernel,
        mesh=mesh,
        out_type=jax.ShapeDtypeStruct((_NW, _B // _NW * _KEEP // 128, 128),
                                      jnp.float32),
        compiler_params=_SC_PARAMS,
        scratch_types=[
            pltpu.VMEM((_GCH, _GSEG), jnp.int32),
            pltpu.VMEM((_RING, _GSEG, 128), jnp.float32),
            pltpu.VMEM((_B // _NW * _KEEP // 128, 128), jnp.float32),
        ] + [pltpu.SemaphoreType.DMA] * _RING,
    )(_sc_gather_body)
    return extract, gather


def kernel(ag_type, trajs, scores):
    extract, gather = _sc_calls()
    table = trajs.reshape(_NSEG, 128)
    xy = extract(table)
    xs = xy[:, 0].reshape(_S, _K, _A)
    ys = xy[:, 1].reshape(_S, _K, _A)
    agt = jnp.swapaxes(ag_type, 1, 2)
    segs, sout = _nms_call(scores, xs, ys, agt)
    scores_k = jnp.swapaxes(sout, 1, 2)
    seg_idx = jnp.transpose(segs, (0, 2, 1)).reshape(_NW, _GCH, _GSEG)
    rows = gather(table, seg_idx)
    trajs_out = rows.reshape(_S, _A, _KP, 16, _C)
    return trajs_out, scores_k
